# baseline (device time: 25287 ns/iter reference)
import jax
import jax.numpy as jnp
from jax import lax
from jax.experimental import pallas as pl
from jax.experimental.pallas import tpu as pltpu

N_Y = 4
MESH = pl.DeviceIdType.MESH


def kernel(x):
    m_per, n = x.shape
    half = m_per // 2

    def body(
        x_ref,
        out_ref,
        yline_ref,
        xbuf_ref,
        rsend_sems,
        rrecv_sems,
        lsend_sems,
        lrecv_sems,
        dsend_sems,
        drecv_sems,
        xsend_sems,
        xrecv_sems,
    ):
        my_x = lax.axis_index("x")
        my_y = lax.axis_index("y")
        my_z = lax.axis_index("z")
        px = 1 - my_x
        edge = (my_y == 0) | (my_y == N_Y - 1)
        far_y = N_Y - 1 - my_y

        barrier_sem = pltpu.get_barrier_semaphore()
        pl.semaphore_signal(
            barrier_sem, inc=1, device_id=(px, my_y, my_z), device_id_type=MESH
        )

        @pl.when(my_y > 0)
        def _():
            pl.semaphore_signal(
                barrier_sem,
                inc=1,
                device_id=(my_x, my_y - 1, my_z),
                device_id_type=MESH,
            )

        @pl.when(my_y < N_Y - 1)
        def _():
            pl.semaphore_signal(
                barrier_sem,
                inc=1,
                device_id=(my_x, my_y + 1, my_z),
                device_id_type=MESH,
            )

        @pl.when(edge)
        def _():
            pl.semaphore_signal(
                barrier_sem,
                inc=1,
                device_id=(my_x, far_y, my_z),
                device_id_type=MESH,
            )

        pl.semaphore_wait(barrier_sem, 3)

        roff = my_x * half
        poff = px * half

        def right_copy(o):
            return pltpu.make_async_remote_copy(
                src_ref=yline_ref.at[o],
                dst_ref=yline_ref.at[o],
                send_sem=rsend_sems.at[o],
                recv_sem=rrecv_sems.at[o],
                device_id=(my_x, my_y + 1, my_z),
                device_id_type=MESH,
            )

        def left_copy(o):
            return pltpu.make_async_remote_copy(
                src_ref=yline_ref.at[o],
                dst_ref=yline_ref.at[o],
                send_sem=lsend_sems.at[o],
                recv_sem=lrecv_sems.at[o],
                device_id=(my_x, my_y - 1, my_z),
                device_id_type=MESH,
            )

        def far_copy(o):
            return pltpu.make_async_remote_copy(
                src_ref=yline_ref.at[o],
                dst_ref=yline_ref.at[o],
                send_sem=dsend_sems.at[0],
                recv_sem=drecv_sems.at[0],
                device_id=(my_x, far_y, my_z),
                device_id_type=MESH,
            )

        def x_copy(o):
            return pltpu.make_async_remote_copy(
                src_ref=yline_ref.at[o],
                dst_ref=xbuf_ref.at[o],
                send_sem=xsend_sems.at[o],
                recv_sem=xrecv_sems.at[o],
                device_id=(px, my_y, my_z),
                device_id_type=MESH,
            )

        yline_ref[pl.ds(my_y, 1)] = x_ref[pl.ds(roff, half), :].astype(
            jnp.bfloat16
        )[None]

        @pl.when(my_y < N_Y - 1)
        def _():
            right_copy(my_y).start()

        @pl.when(my_y > 0)
        def _():
            left_copy(my_y).start()

        @pl.when(edge)
        def _():
            far_copy(my_y).start()

        out_ref[pl.ds(my_y * m_per, m_per), :] = x_ref[:, :].astype(jnp.bfloat16)

        def land(o, fwd):
            if fwd is not None:
                fwd(o)
            x_copy(o).start()
            out_ref[pl.ds(o * m_per + roff, half), :] = yline_ref[pl.ds(o, 1)][0]

        @pl.when(my_y > 0)
        def _():
            o = my_y - 1
            right_copy(o).wait_recv()

            def fwd(o):
                @pl.when(my_y < N_Y - 1)
                def _():
                    right_copy(o).start()

            land(o, fwd)

        @pl.when(my_y < N_Y - 1)
        def _():
            o = my_y + 1
            left_copy(o).wait_recv()

            def fwd(o):
                @pl.when(my_y > 0)
                def _():
                    left_copy(o).start()

            land(o, fwd)

        @pl.when(my_y > 1)
        def _():
            o = my_y - 2
            right_copy(o).wait_recv()
            land(o, None)

        @pl.when(my_y < N_Y - 2)
        def _():
            o = my_y + 2
            left_copy(o).wait_recv()
            land(o, None)

        @pl.when(edge)
        def _():
            far_copy(far_y).wait_recv()
            land(far_y, None)

        def x_land(o):
            x_copy(o).wait_recv()
            out_ref[pl.ds(o * m_per + poff, half), :] = xbuf_ref[pl.ds(o, 1)][0]

        @pl.when(my_y > 0)
        def _():
            x_land(my_y - 1)

        @pl.when(my_y < N_Y - 1)
        def _():
            x_land(my_y + 1)

        @pl.when(my_y > 1)
        def _():
            x_land(my_y - 2)

        @pl.when(my_y < N_Y - 2)
        def _():
            x_land(my_y + 2)

        @pl.when(edge)
        def _():
            x_land(far_y)

        @pl.when(my_y < N_Y - 1)
        def _():
            right_copy(my_y).wait_send()

        @pl.when((my_y > 0) & (my_y < N_Y - 1))
        def _():
            right_copy(my_y - 1).wait_send()
            left_copy(my_y + 1).wait_send()

        @pl.when(my_y > 0)
        def _():
            left_copy(my_y).wait_send()

        @pl.when(edge)
        def _():
            far_copy(my_y).wait_send()

        @pl.when(my_y > 0)
        def _():
            x_copy(my_y - 1).wait_send()

        @pl.when(my_y < N_Y - 1)
        def _():
            x_copy(my_y + 1).wait_send()

        @pl.when(my_y > 1)
        def _():
            x_copy(my_y - 2).wait_send()

        @pl.when(my_y < N_Y - 2)
        def _():
            x_copy(my_y + 2).wait_send()

        @pl.when(edge)
        def _():
            x_copy(far_y).wait_send()

    return pl.pallas_call(
        body,
        out_shape=jax.ShapeDtypeStruct((N_Y * m_per, n), jnp.bfloat16),
        in_specs=[pl.BlockSpec(memory_space=pltpu.VMEM)],
        out_specs=pl.BlockSpec(memory_space=pltpu.VMEM),
        scratch_shapes=[
            pltpu.VMEM((N_Y, half, n), jnp.bfloat16),
            pltpu.VMEM((N_Y, half, n), jnp.bfloat16),
            pltpu.SemaphoreType.DMA((N_Y,)),
            pltpu.SemaphoreType.DMA((N_Y,)),
            pltpu.SemaphoreType.DMA((N_Y,)),
            pltpu.SemaphoreType.DMA((N_Y,)),
            pltpu.SemaphoreType.DMA((1,)),
            pltpu.SemaphoreType.DMA((1,)),
            pltpu.SemaphoreType.DMA((N_Y,)),
            pltpu.SemaphoreType.DMA((N_Y,)),
        ],
        compiler_params=pltpu.CompilerParams(collective_id=0),
    )(x)


# device time: 24492 ns/iter; 1.0325x vs baseline; 1.0325x over previous
import jax
import jax.numpy as jnp
from jax import lax
from jax.experimental import pallas as pl
from jax.experimental.pallas import tpu as pltpu

N_Y = 4
MESH = pl.DeviceIdType.MESH


def kernel(x):
    m_per, n = x.shape
    half = m_per // 2

    def body(
        x_ref,
        out_ref,
        yline_ref,
        xbuf_ref,
        rsend_sems,
        rrecv_sems,
        lsend_sems,
        lrecv_sems,
        xsend_sems,
        xrecv_sems,
    ):
        my_x = lax.axis_index("x")
        my_y = lax.axis_index("y")
        my_z = lax.axis_index("z")
        px = 1 - my_x
        interior = (my_y > 0) & (my_y < N_Y - 1)

        barrier_sem = pltpu.get_barrier_semaphore()
        pl.semaphore_signal(
            barrier_sem, inc=1, device_id=(px, my_y, my_z), device_id_type=MESH
        )

        @pl.when(my_y > 0)
        def _():
            pl.semaphore_signal(
                barrier_sem,
                inc=1,
                device_id=(my_x, my_y - 1, my_z),
                device_id_type=MESH,
            )

        @pl.when(my_y < N_Y - 1)
        def _():
            pl.semaphore_signal(
                barrier_sem,
                inc=1,
                device_id=(my_x, my_y + 1, my_z),
                device_id_type=MESH,
            )

        @pl.when(interior)
        def _():
            pl.semaphore_wait(barrier_sem, 3)

        @pl.when(~interior)
        def _():
            pl.semaphore_wait(barrier_sem, 2)

        roff = my_x * half
        poff = px * half

        def right_copy(o):
            return pltpu.make_async_remote_copy(
                src_ref=yline_ref.at[o],
                dst_ref=yline_ref.at[o],
                send_sem=rsend_sems.at[o],
                recv_sem=rrecv_sems.at[o],
                device_id=(my_x, my_y + 1, my_z),
                device_id_type=MESH,
            )

        def left_copy(o):
            return pltpu.make_async_remote_copy(
                src_ref=yline_ref.at[o],
                dst_ref=yline_ref.at[o],
                send_sem=lsend_sems.at[o],
                recv_sem=lrecv_sems.at[o],
                device_id=(my_x, my_y - 1, my_z),
                device_id_type=MESH,
            )

        def x_copy(o):
            return pltpu.make_async_remote_copy(
                src_ref=yline_ref.at[o],
                dst_ref=xbuf_ref.at[o],
                send_sem=xsend_sems.at[o],
                recv_sem=xrecv_sems.at[o],
                device_id=(px, my_y, my_z),
                device_id_type=MESH,
            )

        yline_ref[pl.ds(my_y, 1)] = x_ref[pl.ds(roff, half), :].astype(
            jnp.bfloat16
        )[None]

        @pl.when(my_y < N_Y - 1)
        def _():
            right_copy(my_y).start()

        @pl.when(my_y > 0)
        def _():
            left_copy(my_y).start()

        out_ref[pl.ds(my_y * m_per, m_per), :] = x_ref[:, :].astype(jnp.bfloat16)

        for d in range(1, N_Y):
            o_r = my_y - d
            o_l = my_y + d

            @pl.when(o_r >= 0)
            def _(o=o_r):
                right_copy(o).wait_recv()

                @pl.when(my_y < N_Y - 1)
                def _():
                    right_copy(o).start()

                x_copy(o).start()
                out_ref[pl.ds(o * m_per + roff, half), :] = yline_ref[
                    pl.ds(o, 1)
                ][0]

            @pl.when(o_l <= N_Y - 1)
            def _(o=o_l):
                left_copy(o).wait_recv()

                @pl.when(my_y > 0)
                def _():
                    left_copy(o).start()

                x_copy(o).start()
                out_ref[pl.ds(o * m_per + roff, half), :] = yline_ref[
                    pl.ds(o, 1)
                ][0]

        for d in range(1, N_Y):
            o_r = my_y - d
            o_l = my_y + d

            @pl.when(o_r >= 0)
            def _(o=o_r):
                x_copy(o).wait_recv()
                out_ref[pl.ds(o * m_per + poff, half), :] = xbuf_ref[
                    pl.ds(o, 1)
                ][0]

            @pl.when(o_l <= N_Y - 1)
            def _(o=o_l):
                x_copy(o).wait_recv()
                out_ref[pl.ds(o * m_per + poff, half), :] = xbuf_ref[
                    pl.ds(o, 1)
                ][0]

        for d in range(0, N_Y):
            o_r = my_y - d
            o_l = my_y + d

            @pl.when((o_r >= 0) & (my_y < N_Y - 1))
            def _(o=o_r):
                right_copy(o).wait_send()

            @pl.when((o_l <= N_Y - 1) & (my_y > 0))
            def _(o=o_l):
                left_copy(o).wait_send()

            @pl.when((d > 0) & (o_r >= 0))
            def _(o=o_r):
                x_copy(o).wait_send()

            @pl.when((d > 0) & (o_l <= N_Y - 1))
            def _(o=o_l):
                x_copy(o).wait_send()

    return pl.pallas_call(
        body,
        out_shape=jax.ShapeDtypeStruct((N_Y * m_per, n), jnp.bfloat16),
        in_specs=[pl.BlockSpec(memory_space=pltpu.VMEM)],
        out_specs=pl.BlockSpec(memory_space=pltpu.VMEM),
        scratch_shapes=[
            pltpu.VMEM((N_Y, half, n), jnp.bfloat16),
            pltpu.VMEM((N_Y, half, n), jnp.bfloat16),
            pltpu.SemaphoreType.DMA((N_Y,)),
            pltpu.SemaphoreType.DMA((N_Y,)),
            pltpu.SemaphoreType.DMA((N_Y,)),
            pltpu.SemaphoreType.DMA((N_Y,)),
            pltpu.SemaphoreType.DMA((N_Y,)),
            pltpu.SemaphoreType.DMA((N_Y,)),
        ],
        compiler_params=pltpu.CompilerParams(collective_id=0),
    )(x)
